# baseline (device time: 1624628 ns/iter reference)
import jax
import jax.numpy as jnp
from jax import lax
from jax.experimental import pallas as pl
from jax.experimental.pallas import tpu as pltpu

N_DEV = 32


def kernel(x, w_mat):
    m_glob, k_loc = x.shape
    _, n = w_mat.shape
    m_loc = m_glob // N_DEV

    def body(x_ref, w_ref, out_ref, comm_ref, send_sems, recv_sems, credit_sem):
        my = lax.axis_index("i")
        left = (my - 1) % N_DEV
        right = (my + 1) % N_DEV

        barrier_sem = pltpu.get_barrier_semaphore()
        for nbr in (left, right):
            pl.semaphore_signal(
                barrier_sem, inc=1,
                device_id=(nbr,), device_id_type=pl.DeviceIdType.MESH,
            )
        pl.semaphore_wait(barrier_sem, 2)

        def partial(c):
            rows = x_ref[pl.ds(c * m_loc, m_loc), :]
            return jnp.dot(rows, w_ref[...], preferred_element_type=jnp.float32)

        for t in range(N_DEV - 1):
            send_slot = t % 2
            recv_slot = (t + 1) % 2
            c_send = (my - 1 - t) % N_DEV
            p = partial(c_send)
            if t == 0:
                comm_ref[send_slot, :, :] = p
            else:
                comm_ref[send_slot, :, :] = comm_ref[send_slot, :, :] + p
            if t > 0:
                pl.semaphore_wait(credit_sem, 1)
            rdma = pltpu.make_async_remote_copy(
                src_ref=comm_ref.at[send_slot],
                dst_ref=comm_ref.at[recv_slot],
                send_sem=send_sems.at[send_slot],
                recv_sem=recv_sems.at[recv_slot],
                device_id=(right,),
                device_id_type=pl.DeviceIdType.MESH,
            )
            rdma.start()
            rdma.wait()
            if t < N_DEV - 2:
                pl.semaphore_signal(
                    credit_sem, inc=1,
                    device_id=(left,), device_id_type=pl.DeviceIdType.MESH,
                )

        final_slot = (N_DEV - 1) % 2
        out_ref[...] = jnp.maximum(
            comm_ref[final_slot, :, :] + partial(my), 0.0
        )

    return pl.pallas_call(
        body,
        out_shape=jax.ShapeDtypeStruct((m_loc, n), jnp.float32),
        in_specs=[
            pl.BlockSpec(memory_space=pltpu.VMEM),
            pl.BlockSpec(memory_space=pltpu.VMEM),
        ],
        out_specs=pl.BlockSpec(memory_space=pltpu.VMEM),
        scratch_shapes=[
            pltpu.VMEM((2, m_loc, n), jnp.float32),
            pltpu.SemaphoreType.DMA((2,)),
            pltpu.SemaphoreType.DMA((2,)),
            pltpu.SemaphoreType.REGULAR,
        ],
        compiler_params=pltpu.CompilerParams(collective_id=0),
    )(x, w_mat)


# device time: 1490880 ns/iter; 1.0897x vs baseline; 1.0897x over previous
import jax
import jax.numpy as jnp
from jax import lax
from jax.experimental import pallas as pl
from jax.experimental.pallas import tpu as pltpu

N_DEV = 32


def kernel(x, w_mat):
    m_glob, k_loc = x.shape
    _, n = w_mat.shape
    m_loc = m_glob // N_DEV
    nh = n // 2

    def body(x_ref, w_ref, out_ref, commR, commL,
             sendR, recvR, sendL, recvL, creditR, creditL):
        my = lax.axis_index("i")
        left = (my - 1) % N_DEV
        right = (my + 1) % N_DEV

        barrier_sem = pltpu.get_barrier_semaphore()
        for nbr in (left, right):
            pl.semaphore_signal(
                barrier_sem, inc=1,
                device_id=(nbr,), device_id_type=pl.DeviceIdType.MESH,
            )
        pl.semaphore_wait(barrier_sem, 2)

        def partial_R(c):
            rows = x_ref[pl.ds(c * m_loc, m_loc), :]
            return jnp.dot(rows, w_ref[:, :nh],
                           preferred_element_type=jnp.float32)

        def partial_L(c):
            rows = x_ref[pl.ds(c * m_loc, m_loc), :]
            return jnp.dot(rows, w_ref[:, nh:],
                           preferred_element_type=jnp.float32)

        def hop(t, s, r, wait_credit, do_signal, last):
            if wait_credit:
                pl.semaphore_wait(creditR, 1)
                pl.semaphore_wait(creditL, 1)
            rdma_r = pltpu.make_async_remote_copy(
                src_ref=commR.at[s], dst_ref=commR.at[r],
                send_sem=sendR.at[s], recv_sem=recvR.at[r],
                device_id=(right,), device_id_type=pl.DeviceIdType.MESH,
            )
            rdma_l = pltpu.make_async_remote_copy(
                src_ref=commL.at[s], dst_ref=commL.at[r],
                send_sem=sendL.at[s], recv_sem=recvL.at[r],
                device_id=(left,), device_id_type=pl.DeviceIdType.MESH,
            )
            rdma_r.start()
            rdma_l.start()
            if last:
                pR = partial_R(my)
                pL = partial_L(my)
            else:
                pR = partial_R((my - 2 - t) % N_DEV)
                pL = partial_L((my + 2 + t) % N_DEV)
            rdma_r.wait()
            rdma_l.wait()
            if last:
                out_ref[:, :nh] = jnp.maximum(commR[r, :, :] + pR, 0.0)
                out_ref[:, nh:] = jnp.maximum(commL[r, :, :] + pL, 0.0)
            else:
                commR[r, :, :] = commR[r, :, :] + pR
                commL[r, :, :] = commL[r, :, :] + pL
            if do_signal:
                pl.semaphore_signal(
                    creditR, inc=1,
                    device_id=(left,), device_id_type=pl.DeviceIdType.MESH,
                )
                pl.semaphore_signal(
                    creditL, inc=1,
                    device_id=(right,), device_id_type=pl.DeviceIdType.MESH,
                )

        commR[0, :, :] = partial_R((my - 1) % N_DEV)
        commL[0, :, :] = partial_L((my + 1) % N_DEV)

        hop(0, 0, 1, wait_credit=False, do_signal=True, last=False)

        def loop_body(k, carry):
            t1 = 1 + 2 * k
            hop(t1, 1, 0, wait_credit=True, do_signal=True, last=False)
            hop(t1 + 1, 0, 1, wait_credit=True, do_signal=True, last=False)
            return carry

        lax.fori_loop(0, (N_DEV - 4) // 2, loop_body, 0)

        hop(N_DEV - 3, 1, 0, wait_credit=True, do_signal=True, last=False)
        hop(N_DEV - 2, 0, 1, wait_credit=True, do_signal=False, last=True)

    return pl.pallas_call(
        body,
        out_shape=jax.ShapeDtypeStruct((m_loc, n), jnp.float32),
        in_specs=[
            pl.BlockSpec(memory_space=pltpu.VMEM),
            pl.BlockSpec(memory_space=pltpu.VMEM),
        ],
        out_specs=pl.BlockSpec(memory_space=pltpu.VMEM),
        scratch_shapes=[
            pltpu.VMEM((2, m_loc, nh), jnp.float32),
            pltpu.VMEM((2, m_loc, nh), jnp.float32),
            pltpu.SemaphoreType.DMA((2,)),
            pltpu.SemaphoreType.DMA((2,)),
            pltpu.SemaphoreType.DMA((2,)),
            pltpu.SemaphoreType.DMA((2,)),
            pltpu.SemaphoreType.REGULAR,
            pltpu.SemaphoreType.REGULAR,
        ],
        compiler_params=pltpu.CompilerParams(collective_id=0),
    )(x, w_mat)


# device time: 805192 ns/iter; 2.0177x vs baseline; 1.8516x over previous
import jax
import jax.numpy as jnp
from jax import lax
from jax.experimental import pallas as pl
from jax.experimental.pallas import tpu as pltpu

N_DEV = 32

_P_PLANE = [(0, 0), (1, 0), (1, 1), (0, 1), (0, 2), (1, 2), (1, 3), (0, 3)]
_LOGICAL_OF = {
    (x, y, z): 8 * z + _P_PLANE.index((x, y))
    for z in range(4)
    for (x, y) in _P_PLANE
}

_HAM_COORDS = []
for _y in range(4):
    _zs = range(4) if _y % 2 == 0 else range(3, -1, -1)
    _HAM_COORDS += [(0, _y, _z) for _z in _zs]
for _y in range(3, -1, -1):
    _zs = range(4) if (3 - _y) % 2 == 0 else range(3, -1, -1)
    _HAM_COORDS += [(1, _y, _z) for _z in _zs]

_HAM = [_LOGICAL_OF[c] for c in _HAM_COORDS]
_POS = [0] * N_DEV
for _p, _l in enumerate(_HAM):
    _POS[_l] = _p


def kernel(x, w_mat):
    m_glob, k_loc = x.shape
    _, n = w_mat.shape
    m_loc = m_glob // N_DEV
    nh = n // 2

    ham = jnp.array(_HAM, dtype=jnp.int32)
    pos_arr = jnp.array(_POS, dtype=jnp.int32)
    my = lax.axis_index("i")
    pos = pos_arr[my]
    t_idx = jnp.arange(N_DEV, dtype=jnp.int32)
    nbrs = jnp.stack(
        [ham[(pos - 1) % N_DEV], ham[(pos + 1) % N_DEV]]
    ).astype(jnp.int32)
    c_r = ham[(pos - 1 - t_idx) % N_DEV]
    c_l = ham[(pos + 1 + t_idx) % N_DEV]

    def body(nbr_ref, cR_ref, cL_ref, x_ref, w_ref, out_ref, commR, commL,
             sendR, recvR, sendL, recvL, creditR, creditL):
        left = nbr_ref[0]
        right = nbr_ref[1]

        barrier_sem = pltpu.get_barrier_semaphore()
        for nbr in (left, right):
            pl.semaphore_signal(
                barrier_sem, inc=1,
                device_id=(nbr,), device_id_type=pl.DeviceIdType.MESH,
            )
        pl.semaphore_wait(barrier_sem, 2)

        def partial_R(c):
            rows = x_ref[pl.ds(c * m_loc, m_loc), :]
            return jnp.dot(rows, w_ref[:, :nh],
                           preferred_element_type=jnp.float32)

        def partial_L(c):
            rows = x_ref[pl.ds(c * m_loc, m_loc), :]
            return jnp.dot(rows, w_ref[:, nh:],
                           preferred_element_type=jnp.float32)

        def hop(t, s, r, wait_credit, do_signal, last):
            if wait_credit:
                pl.semaphore_wait(creditR, 1)
                pl.semaphore_wait(creditL, 1)
            rdma_r = pltpu.make_async_remote_copy(
                src_ref=commR.at[s], dst_ref=commR.at[r],
                send_sem=sendR.at[s], recv_sem=recvR.at[r],
                device_id=(right,), device_id_type=pl.DeviceIdType.MESH,
            )
            rdma_l = pltpu.make_async_remote_copy(
                src_ref=commL.at[s], dst_ref=commL.at[r],
                send_sem=sendL.at[s], recv_sem=recvL.at[r],
                device_id=(left,), device_id_type=pl.DeviceIdType.MESH,
            )
            rdma_r.start()
            rdma_l.start()
            pR = partial_R(cR_ref[t + 1])
            pL = partial_L(cL_ref[t + 1])
            rdma_r.wait()
            rdma_l.wait()
            if last:
                out_ref[:, :nh] = jnp.maximum(commR[r, :, :] + pR, 0.0)
                out_ref[:, nh:] = jnp.maximum(commL[r, :, :] + pL, 0.0)
            else:
                commR[r, :, :] = commR[r, :, :] + pR
                commL[r, :, :] = commL[r, :, :] + pL
            if do_signal:
                pl.semaphore_signal(
                    creditR, inc=1,
                    device_id=(left,), device_id_type=pl.DeviceIdType.MESH,
                )
                pl.semaphore_signal(
                    creditL, inc=1,
                    device_id=(right,), device_id_type=pl.DeviceIdType.MESH,
                )

        commR[0, :, :] = partial_R(cR_ref[0])
        commL[0, :, :] = partial_L(cL_ref[0])

        hop(0, 0, 1, wait_credit=False, do_signal=True, last=False)

        def loop_body(k, carry):
            t1 = 1 + 2 * k
            hop(t1, 1, 0, wait_credit=True, do_signal=True, last=False)
            hop(t1 + 1, 0, 1, wait_credit=True, do_signal=True, last=False)
            return carry

        lax.fori_loop(0, (N_DEV - 4) // 2, loop_body, 0)

        hop(N_DEV - 3, 1, 0, wait_credit=True, do_signal=True, last=False)
        hop(N_DEV - 2, 0, 1, wait_credit=True, do_signal=False, last=True)

    return pl.pallas_call(
        body,
        out_shape=jax.ShapeDtypeStruct((m_loc, n), jnp.float32),
        in_specs=[
            pl.BlockSpec(memory_space=pltpu.SMEM),
            pl.BlockSpec(memory_space=pltpu.SMEM),
            pl.BlockSpec(memory_space=pltpu.SMEM),
            pl.BlockSpec(memory_space=pltpu.VMEM),
            pl.BlockSpec(memory_space=pltpu.VMEM),
        ],
        out_specs=pl.BlockSpec(memory_space=pltpu.VMEM),
        scratch_shapes=[
            pltpu.VMEM((2, m_loc, nh), jnp.float32),
            pltpu.VMEM((2, m_loc, nh), jnp.float32),
            pltpu.SemaphoreType.DMA((2,)),
            pltpu.SemaphoreType.DMA((2,)),
            pltpu.SemaphoreType.DMA((2,)),
            pltpu.SemaphoreType.DMA((2,)),
            pltpu.SemaphoreType.REGULAR,
            pltpu.SemaphoreType.REGULAR,
        ],
        compiler_params=pltpu.CompilerParams(collective_id=0),
    )(nbrs, c_r, c_l, x, w_mat)


# device time: 792551 ns/iter; 2.0499x vs baseline; 1.0159x over previous
import jax
import jax.numpy as jnp
from jax import lax
from jax.experimental import pallas as pl
from jax.experimental.pallas import tpu as pltpu

N_DEV = 32

_P_PLANE = [(0, 0), (1, 0), (1, 1), (0, 1), (0, 2), (1, 2), (1, 3), (0, 3)]
_LOGICAL_OF = {
    (x, y, z): 8 * z + _P_PLANE.index((x, y))
    for z in range(4)
    for (x, y) in _P_PLANE
}

_HAM_COORDS = []
for _y in range(4):
    _zs = range(4) if _y % 2 == 0 else range(3, -1, -1)
    _HAM_COORDS += [(0, _y, _z) for _z in _zs]
for _y in range(3, -1, -1):
    _zs = range(4) if (3 - _y) % 2 == 0 else range(3, -1, -1)
    _HAM_COORDS += [(1, _y, _z) for _z in _zs]

_HAM = [_LOGICAL_OF[c] for c in _HAM_COORDS]
_POS = [0] * N_DEV
for _p, _l in enumerate(_HAM):
    _POS[_l] = _p


def kernel(x, w_mat):
    m_glob, k_loc = x.shape
    _, n = w_mat.shape
    m_loc = m_glob // N_DEV
    nh = n // 2

    ham = jnp.array(_HAM, dtype=jnp.int32)
    pos_arr = jnp.array(_POS, dtype=jnp.int32)
    my = lax.axis_index("i")
    pos = pos_arr[my]
    t_idx = jnp.arange(N_DEV, dtype=jnp.int32)
    nbrs = jnp.stack(
        [ham[(pos - 1) % N_DEV], ham[(pos + 1) % N_DEV]]
    ).astype(jnp.int32)
    c_r = ham[(pos - 1 - t_idx) % N_DEV]
    c_l = ham[(pos + 1 + t_idx) % N_DEV]

    def body(nbr_ref, cR_ref, cL_ref, x_ref, w_ref, out_ref, commR, commL,
             sendR, recvR, sendL, recvL, creditR, creditL):
        left = nbr_ref[0]
        right = nbr_ref[1]

        barrier_sem = pltpu.get_barrier_semaphore()
        for nbr in (left, right):
            pl.semaphore_signal(
                barrier_sem, inc=1,
                device_id=(nbr,), device_id_type=pl.DeviceIdType.MESH,
            )
        pl.semaphore_wait(barrier_sem, 2)

        def partial_R(c):
            rows = x_ref[pl.ds(c * m_loc, m_loc), :]
            return jnp.dot(rows, w_ref[:, :nh],
                           preferred_element_type=jnp.float32)

        def partial_L(c):
            rows = x_ref[pl.ds(c * m_loc, m_loc), :]
            return jnp.dot(rows, w_ref[:, nh:],
                           preferred_element_type=jnp.float32)

        def hop(t, s, r, wait_credit, do_signal, last):
            if wait_credit:
                pl.semaphore_wait(creditR, 1)
                pl.semaphore_wait(creditL, 1)
            rdma_r = pltpu.make_async_remote_copy(
                src_ref=commR.at[s], dst_ref=commR.at[r],
                send_sem=sendR.at[s], recv_sem=recvR.at[r],
                device_id=(right,), device_id_type=pl.DeviceIdType.MESH,
            )
            rdma_l = pltpu.make_async_remote_copy(
                src_ref=commL.at[s], dst_ref=commL.at[r],
                send_sem=sendL.at[s], recv_sem=recvL.at[r],
                device_id=(left,), device_id_type=pl.DeviceIdType.MESH,
            )
            rdma_r.start()
            rdma_l.start()
            pR = partial_R(cR_ref[t + 1])
            pL = partial_L(cL_ref[t + 1])
            rdma_r.wait()
            if do_signal:
                pl.semaphore_signal(
                    creditR, inc=1,
                    device_id=(left,), device_id_type=pl.DeviceIdType.MESH,
                )
            rdma_l.wait()
            if do_signal:
                pl.semaphore_signal(
                    creditL, inc=1,
                    device_id=(right,), device_id_type=pl.DeviceIdType.MESH,
                )
            if last:
                out_ref[:, :nh] = jnp.maximum(commR[r, :, :] + pR, 0.0)
                out_ref[:, nh:] = jnp.maximum(commL[r, :, :] + pL, 0.0)
            else:
                commR[r, :, :] = commR[r, :, :] + pR
                commL[r, :, :] = commL[r, :, :] + pL

        commR[0, :, :] = partial_R(cR_ref[0])
        commL[0, :, :] = partial_L(cL_ref[0])

        hop(0, 0, 1, wait_credit=False, do_signal=True, last=False)

        def loop_body(k, carry):
            t1 = 1 + 2 * k
            hop(t1, 1, 0, wait_credit=True, do_signal=True, last=False)
            hop(t1 + 1, 0, 1, wait_credit=True, do_signal=True, last=False)
            return carry

        lax.fori_loop(0, (N_DEV - 4) // 2, loop_body, 0)

        hop(N_DEV - 3, 1, 0, wait_credit=True, do_signal=True, last=False)
        hop(N_DEV - 2, 0, 1, wait_credit=True, do_signal=False, last=True)

    return pl.pallas_call(
        body,
        out_shape=jax.ShapeDtypeStruct((m_loc, n), jnp.float32),
        in_specs=[
            pl.BlockSpec(memory_space=pltpu.SMEM),
            pl.BlockSpec(memory_space=pltpu.SMEM),
            pl.BlockSpec(memory_space=pltpu.SMEM),
            pl.BlockSpec(memory_space=pltpu.VMEM),
            pl.BlockSpec(memory_space=pltpu.VMEM),
        ],
        out_specs=pl.BlockSpec(memory_space=pltpu.VMEM),
        scratch_shapes=[
            pltpu.VMEM((2, m_loc, nh), jnp.float32),
            pltpu.VMEM((2, m_loc, nh), jnp.float32),
            pltpu.SemaphoreType.DMA((2,)),
            pltpu.SemaphoreType.DMA((2,)),
            pltpu.SemaphoreType.DMA((2,)),
            pltpu.SemaphoreType.DMA((2,)),
            pltpu.SemaphoreType.REGULAR,
            pltpu.SemaphoreType.REGULAR,
        ],
        compiler_params=pltpu.CompilerParams(collective_id=0),
    )(nbrs, c_r, c_l, x, w_mat)


# device time: 740469 ns/iter; 2.1941x vs baseline; 1.0703x over previous
import jax
import jax.numpy as jnp
from jax import lax
from jax.experimental import pallas as pl
from jax.experimental.pallas import tpu as pltpu

N_DEV = 32

_P_PLANE = [(0, 0), (1, 0), (1, 1), (0, 1), (0, 2), (1, 2), (1, 3), (0, 3)]
_LOGICAL_OF = {
    (x, y, z): 8 * z + _P_PLANE.index((x, y))
    for z in range(4)
    for (x, y) in _P_PLANE
}

_HAM_COORDS = []
for _y in range(4):
    _zs = range(4) if _y % 2 == 0 else range(3, -1, -1)
    _HAM_COORDS += [(0, _y, _z) for _z in _zs]
for _y in range(3, -1, -1):
    _zs = range(4) if (3 - _y) % 2 == 0 else range(3, -1, -1)
    _HAM_COORDS += [(1, _y, _z) for _z in _zs]

_HAM = [_LOGICAL_OF[c] for c in _HAM_COORDS]
_POS = [0] * N_DEV
for _p, _l in enumerate(_HAM):
    _POS[_l] = _p


def kernel(x, w_mat):
    m_glob, k_loc = x.shape
    _, n = w_mat.shape
    m_loc = m_glob // N_DEV
    nh = n // 2
    nq = nh // 2

    ham = jnp.array(_HAM, dtype=jnp.int32)
    pos_arr = jnp.array(_POS, dtype=jnp.int32)
    my = lax.axis_index("i")
    pos = pos_arr[my]
    t_idx = jnp.arange(N_DEV, dtype=jnp.int32)
    nbrs = jnp.stack(
        [ham[(pos - 1) % N_DEV], ham[(pos + 1) % N_DEV]]
    ).astype(jnp.int32)
    c_r = ham[(pos - 1 - t_idx) % N_DEV]
    c_l = ham[(pos + 1 + t_idx) % N_DEV]

    SUBS = ((0, 0), (1, nq))

    def body(nbr_ref, cR_ref, cL_ref, x_ref, w_ref, out_ref,
             commR, commL, pstageR, pstageL,
             sendR, recvR, sendL, recvL,
             credR0, credR1, credL0, credL1):
        left = nbr_ref[0]
        right = nbr_ref[1]
        credR = (credR0, credR1)
        credL = (credL0, credL1)

        barrier_sem = pltpu.get_barrier_semaphore()
        for nbr in (left, right):
            pl.semaphore_signal(
                barrier_sem, inc=1,
                device_id=(nbr,), device_id_type=pl.DeviceIdType.MESH,
            )
        pl.semaphore_wait(barrier_sem, 2)

        def partial_R(c):
            rows = x_ref[pl.ds(c * m_loc, m_loc), :]
            return jnp.dot(rows, w_ref[:, :nh],
                           preferred_element_type=jnp.float32)

        def partial_L(c):
            rows = x_ref[pl.ds(c * m_loc, m_loc), :]
            return jnp.dot(rows, w_ref[:, nh:],
                           preferred_element_type=jnp.float32)

        def mk(comm, slot_src, slot_dst, off, ssem, rsem, dev):
            return pltpu.make_async_remote_copy(
                src_ref=comm.at[slot_src, :, pl.ds(off, nq)],
                dst_ref=comm.at[slot_dst, :, pl.ds(off, nq)],
                send_sem=ssem, recv_sem=rsem,
                device_id=(dev,), device_id_type=pl.DeviceIdType.MESH,
            )

        def hop(t, s, r, first, last):
            started = []
            for sub, off in SUBS:
                for comm, pstage, cred, recvs, sends, dev in (
                    (commR, pstageR, credR, recvR, sendR, right),
                    (commL, pstageL, credL, recvL, sendL, left),
                ):
                    if not first:
                        mk(comm, s, s, off, sends.at[s, sub],
                           recvs.at[s, sub], dev).wait_recv()
                        comm[s, :, off:off + nq] = (
                            comm[s, :, off:off + nq] + pstage[:, off:off + nq]
                        )
                        pl.semaphore_wait(cred[sub], 1)
                    rdma = mk(comm, s, r, off, sends.at[s, sub],
                              recvs.at[r, sub], dev)
                    rdma.start()
                    started.append((rdma, cred[sub], dev))
            pstageR[:, :] = partial_R(cR_ref[t + 1])
            pstageL[:, :] = partial_L(cL_ref[t + 1])
            for rdma, cred, dev in started:
                rdma.wait_send()
                if not last:
                    pl.semaphore_signal(
                        cred, inc=1,
                        device_id=(left if dev is right else right,),
                        device_id_type=pl.DeviceIdType.MESH,
                    )

        commR[0, :, :] = partial_R(cR_ref[0])
        commL[0, :, :] = partial_L(cL_ref[0])

        hop(0, 0, 1, first=True, last=False)

        def loop_body(k, carry):
            t1 = 1 + 2 * k
            hop(t1, 1, 0, first=False, last=False)
            hop(t1 + 1, 0, 1, first=False, last=False)
            return carry

        lax.fori_loop(0, (N_DEV - 4) // 2, loop_body, 0)

        hop(N_DEV - 3, 1, 0, first=False, last=False)
        hop(N_DEV - 2, 0, 1, first=False, last=True)

        for sub, off in SUBS:
            mk(commR, 1, 1, off, sendR.at[1, sub],
               recvR.at[1, sub], right).wait_recv()
            out_ref[:, off:off + nq] = jnp.maximum(
                commR[1, :, off:off + nq] + pstageR[:, off:off + nq], 0.0
            )
            mk(commL, 1, 1, off, sendL.at[1, sub],
               recvL.at[1, sub], left).wait_recv()
            out_ref[:, nh + off:nh + off + nq] = jnp.maximum(
                commL[1, :, off:off + nq] + pstageL[:, off:off + nq], 0.0
            )

    return pl.pallas_call(
        body,
        out_shape=jax.ShapeDtypeStruct((m_loc, n), jnp.float32),
        in_specs=[
            pl.BlockSpec(memory_space=pltpu.SMEM),
            pl.BlockSpec(memory_space=pltpu.SMEM),
            pl.BlockSpec(memory_space=pltpu.SMEM),
            pl.BlockSpec(memory_space=pltpu.VMEM),
            pl.BlockSpec(memory_space=pltpu.VMEM),
        ],
        out_specs=pl.BlockSpec(memory_space=pltpu.VMEM),
        scratch_shapes=[
            pltpu.VMEM((2, m_loc, nh), jnp.float32),
            pltpu.VMEM((2, m_loc, nh), jnp.float32),
            pltpu.VMEM((m_loc, nh), jnp.float32),
            pltpu.VMEM((m_loc, nh), jnp.float32),
            pltpu.SemaphoreType.DMA((2, 2)),
            pltpu.SemaphoreType.DMA((2, 2)),
            pltpu.SemaphoreType.DMA((2, 2)),
            pltpu.SemaphoreType.DMA((2, 2)),
            pltpu.SemaphoreType.REGULAR,
            pltpu.SemaphoreType.REGULAR,
            pltpu.SemaphoreType.REGULAR,
            pltpu.SemaphoreType.REGULAR,
        ],
        compiler_params=pltpu.CompilerParams(collective_id=0),
    )(nbrs, c_r, c_l, x, w_mat)


# device time: 740428 ns/iter; 2.1942x vs baseline; 1.0001x over previous
import jax
import jax.numpy as jnp
from jax import lax
from jax.experimental import pallas as pl
from jax.experimental.pallas import tpu as pltpu

N_DEV = 32

_P_PLANE = [(0, 0), (1, 0), (1, 1), (0, 1), (0, 2), (1, 2), (1, 3), (0, 3)]
_LOGICAL_OF = {
    (x, y, z): 8 * z + _P_PLANE.index((x, y))
    for z in range(4)
    for (x, y) in _P_PLANE
}

_HAM_COORDS = []
for _y in range(4):
    _zs = range(4) if _y % 2 == 0 else range(3, -1, -1)
    _HAM_COORDS += [(0, _y, _z) for _z in _zs]
for _y in range(3, -1, -1):
    _zs = range(4) if (3 - _y) % 2 == 0 else range(3, -1, -1)
    _HAM_COORDS += [(1, _y, _z) for _z in _zs]

_HAM = [_LOGICAL_OF[c] for c in _HAM_COORDS]
_POS = [0] * N_DEV
for _p, _l in enumerate(_HAM):
    _POS[_l] = _p


def kernel(x, w_mat):
    m_glob, k_loc = x.shape
    _, n = w_mat.shape
    m_loc = m_glob // N_DEV
    nh = n // 2
    nq = nh // 2

    ham = jnp.array(_HAM, dtype=jnp.int32)
    pos_arr = jnp.array(_POS, dtype=jnp.int32)
    my = lax.axis_index("i")
    pos = pos_arr[my]
    t_idx = jnp.arange(N_DEV, dtype=jnp.int32)
    nbrs = jnp.stack(
        [ham[(pos - 1) % N_DEV], ham[(pos + 1) % N_DEV]]
    ).astype(jnp.int32)
    c_r = ham[(pos - 1 - t_idx) % N_DEV]
    c_l = ham[(pos + 1 + t_idx) % N_DEV]

    SUBS = ((0, 0), (1, nq))

    def body(nbr_ref, cR_ref, cL_ref, x_ref, w_ref, out_ref,
             commR, commL, pstageR, pstageL,
             sendR, recvR, sendL, recvL,
             credR0, credR1, credL0, credL1):
        left = nbr_ref[0]
        right = nbr_ref[1]
        credR = (credR0, credR1)
        credL = (credL0, credL1)

        barrier_sem = pltpu.get_barrier_semaphore()
        for nbr in (left, right):
            pl.semaphore_signal(
                barrier_sem, inc=1,
                device_id=(nbr,), device_id_type=pl.DeviceIdType.MESH,
            )
        pl.semaphore_wait(barrier_sem, 2)

        def partial_R(c):
            rows = x_ref[pl.ds(c * m_loc, m_loc), :]
            return jnp.dot(rows, w_ref[:, :nh],
                           preferred_element_type=jnp.float32)

        def partial_L(c):
            rows = x_ref[pl.ds(c * m_loc, m_loc), :]
            return jnp.dot(rows, w_ref[:, nh:],
                           preferred_element_type=jnp.float32)

        def mk(comm, slot_src, slot_dst, off, ssem, rsem, dev):
            return pltpu.make_async_remote_copy(
                src_ref=comm.at[slot_src, :, pl.ds(off, nq)],
                dst_ref=comm.at[slot_dst, :, pl.ds(off, nq)],
                send_sem=ssem, recv_sem=rsem,
                device_id=(dev,), device_id_type=pl.DeviceIdType.MESH,
            )

        def hop(t, s, r, first, last):
            started = []
            for sub, off in SUBS:
                for comm, pstage, cred, recvs, sends, dev in (
                    (commR, pstageR, credR, recvR, sendR, right),
                    (commL, pstageL, credL, recvL, sendL, left),
                ):
                    if not first:
                        mk(comm, s, s, off, sends.at[s, sub],
                           recvs.at[s, sub], dev).wait_recv()
                        comm[s, :, off:off + nq] = (
                            comm[s, :, off:off + nq] + pstage[:, off:off + nq]
                        )
                        pl.semaphore_wait(cred[sub], 1)
                    rdma = mk(comm, s, r, off, sends.at[s, sub],
                              recvs.at[r, sub], dev)
                    rdma.start()
                    started.append((rdma, cred[sub], dev))
            pstageR[:, :] = partial_R(cR_ref[t + 1])
            pstageL[:, :] = partial_L(cL_ref[t + 1])
            for rdma, cred, dev in started:
                rdma.wait_send()
                if not last:
                    pl.semaphore_signal(
                        cred, inc=1,
                        device_id=(left if dev is right else right,),
                        device_id_type=pl.DeviceIdType.MESH,
                    )

        commR[0, :, :] = partial_R(cR_ref[0])
        commL[0, :, :] = partial_L(cL_ref[0])

        hop(0, 0, 1, first=True, last=False)

        def loop_body(t, carry):
            s = lax.rem(t, 2)
            hop(t, s, 1 - s, first=False, last=False)
            return carry

        lax.fori_loop(1, N_DEV - 2, loop_body, 0)

        hop(N_DEV - 2, 0, 1, first=False, last=True)

        for sub, off in SUBS:
            mk(commR, 1, 1, off, sendR.at[1, sub],
               recvR.at[1, sub], right).wait_recv()
            out_ref[:, off:off + nq] = jnp.maximum(
                commR[1, :, off:off + nq] + pstageR[:, off:off + nq], 0.0
            )
            mk(commL, 1, 1, off, sendL.at[1, sub],
               recvL.at[1, sub], left).wait_recv()
            out_ref[:, nh + off:nh + off + nq] = jnp.maximum(
                commL[1, :, off:off + nq] + pstageL[:, off:off + nq], 0.0
            )

    return pl.pallas_call(
        body,
        out_shape=jax.ShapeDtypeStruct((m_loc, n), jnp.float32),
        in_specs=[
            pl.BlockSpec(memory_space=pltpu.SMEM),
            pl.BlockSpec(memory_space=pltpu.SMEM),
            pl.BlockSpec(memory_space=pltpu.SMEM),
            pl.BlockSpec(memory_space=pltpu.VMEM),
            pl.BlockSpec(memory_space=pltpu.VMEM),
        ],
        out_specs=pl.BlockSpec(memory_space=pltpu.VMEM),
        scratch_shapes=[
            pltpu.VMEM((2, m_loc, nh), jnp.float32),
            pltpu.VMEM((2, m_loc, nh), jnp.float32),
            pltpu.VMEM((m_loc, nh), jnp.float32),
            pltpu.VMEM((m_loc, nh), jnp.float32),
            pltpu.SemaphoreType.DMA((2, 2)),
            pltpu.SemaphoreType.DMA((2, 2)),
            pltpu.SemaphoreType.DMA((2, 2)),
            pltpu.SemaphoreType.DMA((2, 2)),
            pltpu.SemaphoreType.REGULAR,
            pltpu.SemaphoreType.REGULAR,
            pltpu.SemaphoreType.REGULAR,
            pltpu.SemaphoreType.REGULAR,
        ],
        compiler_params=pltpu.CompilerParams(collective_id=0),
    )(nbrs, c_r, c_l, x, w_mat)
